# baseline (speedup 1.0000x reference)
import jax
import jax.numpy as jnp
from jax import lax
from jax.experimental import pallas as pl
from jax.experimental.pallas import tpu as pltpu
from jax.experimental.pallas import tpu_sc as plsc

NC = 2
NS = 16
NW = NC * NS
CHUNK = 128
NCHUNK = 126
PER_W = CHUNK * NCHUNK
PAD_TOTAL = NW * PER_W
GROUPS = PER_W // 16

LN2 = 0.6931471805599453
_C = (0.01477872, -0.18029977, 0.9618663, -2.9452062,
      5.7234015, -7.443873, 7.1100354, -3.2407022)


def _sc_body(nch, probs, states, choices, weights, ll_out, w_out,
             st_v, ch_v, wt_v, vals_v, acc_v, sem_in, gsem):
    wid = lax.axis_index("s") * NC + lax.axis_index("c")
    base = wid * PER_W

    cp_st = pltpu.async_copy(states.at[pl.ds(base, PER_W)], st_v, sem_in)
    cp_ch = pltpu.async_copy(choices.at[pl.ds(base, PER_W)], ch_v, sem_in)
    cp_wt = pltpu.async_copy(weights.at[pl.ds(base, PER_W)], wt_v, sem_in)
    cp_st.wait()
    cp_ch.wait()
    cp_wt.wait()

    def idx_body(i, _):
        st_v[pl.ds(i * 16, 16)] = (
            st_v[pl.ds(i * 16, 16)] * nch + ch_v[pl.ds(i * 16, 16)])
        return 0
    lax.fori_loop(0, GROUPS, idx_body, 0, unroll=8)

    def fire_body(j, _):
        pltpu.async_copy(
            probs.at[st_v.at[pl.ds(j * CHUNK, CHUNK)]],
            vals_v.at[pl.ds(j * CHUNK, CHUNK)], gsem)
        return 0
    lax.fori_loop(0, NCHUNK, fire_body, 0)

    def comp_body(j, carry):
        acc_ll, acc_w = carry
        pltpu.make_async_copy(
            probs.at[st_v.at[pl.ds(j * CHUNK, CHUNK)]],
            vals_v.at[pl.ds(j * CHUNK, CHUNK)], gsem).wait()
        cbase = j * CHUNK
        for g in range(CHUNK // 16):
            p = vals_v[pl.ds(cbase + g * 16, 16)]
            w = wt_v[pl.ds(cbase + g * 16, 16)]
            p = jnp.minimum(jnp.maximum(p, jnp.float32(1e-10)),
                            jnp.float32(1.0))
            bits = lax.bitcast_convert_type(p, jnp.int32)
            e = (bits >> 23) - 127
            f = lax.bitcast_convert_type(
                (bits & 0x7FFFFF) | 0x3F800000, jnp.float32)
            poly = jnp.float32(_C[0])
            for c in _C[1:]:
                poly = poly * f + jnp.float32(c)
            acc_ll = acc_ll + (e.astype(jnp.float32) + poly) * w
            acc_w = acc_w + w
        return acc_ll, acc_w

    zeros = jnp.zeros((16,), jnp.float32)
    acc_ll, acc_w = lax.fori_loop(0, NCHUNK, comp_body, (zeros, zeros))

    acc_v[...] = acc_ll
    pltpu.sync_copy(acc_v, ll_out.at[wid])
    acc_v[...] = acc_w
    pltpu.sync_copy(acc_v, w_out.at[wid])


def _make_sc_call(nch):
    mesh = plsc.VectorSubcoreMesh(
        core_axis_name="c", subcore_axis_name="s",
        num_cores=NC, num_subcores=NS)
    return pl.kernel(
        lambda *args: _sc_body(nch, *args),
        out_type=[
            jax.ShapeDtypeStruct((NW, 16), jnp.float32),
            jax.ShapeDtypeStruct((NW, 16), jnp.float32),
        ],
        mesh=mesh,
        scratch_types=[
            pltpu.VMEM((PER_W,), jnp.int32),
            pltpu.VMEM((PER_W,), jnp.int32),
            pltpu.VMEM((PER_W,), jnp.float32),
            pltpu.VMEM((PER_W,), jnp.float32),
            pltpu.VMEM((16,), jnp.float32),
            pltpu.SemaphoreType.DMA,
            pltpu.SemaphoreType.DMA,
        ],
    )


def kernel(choice_probs, obs_states, obs_choices, obs_weights):
    n_states, n_choices = choice_probs.shape
    n_obs = obs_states.shape[0]
    npad = PAD_TOTAL - n_obs
    pad_states = jnp.arange(npad, dtype=jnp.int32) % n_states
    states_p = jnp.concatenate([obs_states.astype(jnp.int32), pad_states])
    choices_p = jnp.concatenate(
        [obs_choices.astype(jnp.int32), jnp.zeros((npad,), jnp.int32)])
    weights_p = jnp.concatenate(
        [obs_weights, jnp.zeros((npad,), jnp.float32)])

    probs_flat = choice_probs.reshape(-1)
    ll_parts, w_parts = _make_sc_call(n_choices)(
        probs_flat, states_p, choices_p, weights_p)

    ll = jnp.sum(ll_parts) * jnp.float32(LN2)
    sw = jnp.sum(w_parts)
    nll = -(ll / sw)
    return jnp.where(jnp.isfinite(nll), nll,
                     jnp.array(1e10, dtype=nll.dtype))
